# Initial kernel scaffold; baseline (speedup 1.0000x reference)
import functools

import jax
import jax.numpy as jnp
from jax import lax
from jax.experimental import pallas as pl
from jax.experimental.pallas import tpu as pltpu
from jax.experimental.pallas import tpu_sc as plsc

_EMB = 64
_NW = 32
_CHUNK = 128


@functools.partial(jax.jit, static_argnames=("n",))
def _sc_gather(idx, table, n):
    nchunk = n // (_NW * _CHUNK)
    mesh = plsc.VectorSubcoreMesh(core_axis_name="c", subcore_axis_name="s")

    @functools.partial(
        pl.kernel,
        out_type=jax.ShapeDtypeStruct((n, _EMB), jnp.float32),
        mesh=mesh,
        scratch_types=[
            pltpu.VMEM((nchunk, _CHUNK), jnp.int32),
            pltpu.VMEM((_CHUNK, _EMB), jnp.float32),
            pltpu.SemaphoreType.DMA,
        ],
    )
    def k(idx_hbm, table_hbm, out_hbm, idx_v, rows_v, sem):
        wid = lax.axis_index("s") * 2 + lax.axis_index("c")
        base = wid * (nchunk * _CHUNK)
        pltpu.sync_copy(idx_hbm.at[wid], idx_v)

        @pl.loop(0, nchunk)
        def _(j):
            pltpu.async_copy(table_hbm.at[idx_v.at[j]], rows_v, sem).wait()
            pltpu.sync_copy(rows_v, out_hbm.at[pl.ds(base + j * _CHUNK, _CHUNK)])

    return k(idx, table)


def kernel(input, table):
    seq, batch = input.shape
    n = seq * batch
    idx = input.reshape(_NW, n // (_NW * _CHUNK), _CHUNK).astype(jnp.int32)
    out = _sc_gather(idx, table.astype(jnp.float32), n)
    return out.reshape(seq, batch, _EMB)


# speedup vs baseline: 2.9475x; 2.9475x over previous
import functools

import jax
import jax.numpy as jnp
from jax import lax
from jax.experimental import pallas as pl
from jax.experimental.pallas import tpu as pltpu
from jax.experimental.pallas import tpu_sc as plsc

_EMB = 64
_NW = 32
_CHUNK = 128


@functools.partial(jax.jit, static_argnames=("n",))
def _sc_gather(idx, table, n):
    nchunk = n // (_NW * _CHUNK)
    mesh = plsc.VectorSubcoreMesh(core_axis_name="c", subcore_axis_name="s")

    @functools.partial(
        pl.kernel,
        out_type=jax.ShapeDtypeStruct((n, _EMB), jnp.float32),
        mesh=mesh,
        scratch_types=[
            pltpu.VMEM((nchunk, _CHUNK), jnp.int32),
            pltpu.VMEM((_CHUNK, _EMB), jnp.float32),
            pltpu.SemaphoreType.DMA,
        ],
        compiler_params=pltpu.CompilerParams(use_tc_tiling_on_sc=False),
    )
    def k(idx_hbm, table_hbm, out_hbm, idx_v, rows_v, sem):
        wid = lax.axis_index("s") * 2 + lax.axis_index("c")
        base = wid * (nchunk * _CHUNK)
        pltpu.sync_copy(idx_hbm.at[wid], idx_v)

        @pl.loop(0, nchunk)
        def _(j):
            pltpu.async_copy(table_hbm.at[idx_v.at[j]], rows_v, sem).wait()
            pltpu.sync_copy(rows_v, out_hbm.at[pl.ds(base + j * _CHUNK, _CHUNK)])

    return k(idx, table)


def kernel(input, table):
    seq, batch = input.shape
    n = seq * batch
    idx = input.reshape(_NW, n // (_NW * _CHUNK), _CHUNK).astype(jnp.int32)
    out = _sc_gather(idx, table.astype(jnp.float32), n)
    return out.reshape(seq, batch, _EMB)


# speedup vs baseline: 3.3262x; 1.1285x over previous
import functools

import jax
import jax.numpy as jnp
from jax import lax
from jax.experimental import pallas as pl
from jax.experimental.pallas import tpu as pltpu
from jax.experimental.pallas import tpu_sc as plsc

_EMB = 64
_NW = 32
_CHUNK = 128
_G_CH = 5
_G_ROWS = _G_CH * _CHUNK


@functools.partial(jax.jit, static_argnames=("n",))
def _sc_gather(idx, table, n):
    nchunk = n // (_NW * _CHUNK)
    ng = nchunk // _G_CH
    assert ng * _G_CH == nchunk and ng >= 4 and ng % 2 == 0
    per_w = nchunk * _CHUNK
    mesh = plsc.VectorSubcoreMesh(core_axis_name="c", subcore_axis_name="s")

    @functools.partial(
        pl.kernel,
        out_type=jax.ShapeDtypeStruct((n, _EMB), jnp.float32),
        mesh=mesh,
        scratch_types=[
            pltpu.VMEM((nchunk, _CHUNK), jnp.int32),
            pltpu.VMEM((_G_ROWS, _EMB), jnp.float32),
            pltpu.VMEM((_G_ROWS, _EMB), jnp.float32),
            pltpu.SemaphoreType.DMA,
            pltpu.SemaphoreType.DMA,
            pltpu.SemaphoreType.DMA,
            pltpu.SemaphoreType.DMA,
        ],
        compiler_params=pltpu.CompilerParams(use_tc_tiling_on_sc=False),
    )
    def k(idx_hbm, table_hbm, out_hbm, idx_v, buf_a, buf_b, gs_a, gs_b, ws_a, ws_b):
        wid = lax.axis_index("s") * 2 + lax.axis_index("c")
        base = wid * per_w
        pltpu.sync_copy(idx_hbm.at[wid], idx_v)

        def issue_gathers(g, buf, sem):
            for c in range(_G_CH):
                pltpu.async_copy(
                    table_hbm.at[idx_v.at[g * _G_CH + c]],
                    buf.at[pl.ds(c * _CHUNK, _CHUNK)],
                    sem,
                )

        def wait_gathers(buf, sem):
            pltpu.make_async_copy(out_hbm.at[pl.ds(0, _G_ROWS)], buf, sem).wait()

        def issue_write(g, buf, sem):
            pltpu.async_copy(buf, out_hbm.at[pl.ds(base + g * _G_ROWS, _G_ROWS)], sem)

        def wait_write(buf, sem):
            pltpu.make_async_copy(buf, out_hbm.at[pl.ds(0, _G_ROWS)], sem).wait()

        issue_gathers(0, buf_a, gs_a)
        issue_gathers(1, buf_b, gs_b)

        @pl.loop(0, ng // 2 - 1)
        def _(kk):
            g0 = 2 * kk
            wait_gathers(buf_a, gs_a)
            issue_write(g0, buf_a, ws_a)
            wait_gathers(buf_b, gs_b)
            issue_write(g0 + 1, buf_b, ws_b)
            wait_write(buf_a, ws_a)
            issue_gathers(g0 + 2, buf_a, gs_a)
            wait_write(buf_b, ws_b)
            issue_gathers(g0 + 3, buf_b, gs_b)

        wait_gathers(buf_a, gs_a)
        issue_write(ng - 2, buf_a, ws_a)
        wait_gathers(buf_b, gs_b)
        issue_write(ng - 1, buf_b, ws_b)
        wait_write(buf_a, ws_a)
        wait_write(buf_b, ws_b)

    return k(idx, table)


def kernel(input, table):
    seq, batch = input.shape
    n = seq * batch
    idx = input.reshape(_NW, n // (_NW * _CHUNK), _CHUNK).astype(jnp.int32)
    out = _sc_gather(idx, table.astype(jnp.float32), n)
    return out.reshape(seq, batch, _EMB)


# speedup vs baseline: 4.2935x; 1.2908x over previous
import functools

import jax
import jax.numpy as jnp
from jax import lax
from jax.experimental import pallas as pl
from jax.experimental.pallas import tpu as pltpu
from jax.experimental.pallas import tpu_sc as plsc

_EMB = 64
_EMBP = 128
_NW = 32
_CHUNK = 128
_NBUF = 5


@functools.partial(jax.jit, static_argnames=("n",))
def _sc_gather(idx, table, n):
    nchunk_w = n // (_NW * _CHUNK)
    assert nchunk_w % _NBUF == 0 and nchunk_w >= 2 * _NBUF
    mesh = plsc.VectorSubcoreMesh(core_axis_name="c", subcore_axis_name="s")

    @functools.partial(
        pl.kernel,
        out_type=jax.ShapeDtypeStruct((n, _EMBP), jnp.float32),
        mesh=mesh,
        scratch_types=[
            pltpu.VMEM((nchunk_w, _CHUNK), jnp.int32),
            [pltpu.VMEM((_CHUNK, _EMBP), jnp.float32)] * _NBUF,
            [pltpu.SemaphoreType.DMA] * _NBUF,
            [pltpu.SemaphoreType.DMA] * _NBUF,
        ],
        compiler_params=pltpu.CompilerParams(use_tc_tiling_on_sc=False),
    )
    def k(idx_hbm, table_hbm, out_hbm, idx_v, bufs, gs, ws):
        wid = lax.axis_index("s") * 2 + lax.axis_index("c")
        base = wid * (nchunk_w * _CHUNK)
        pltpu.sync_copy(idx_hbm.at[pl.ds(wid * nchunk_w, nchunk_w)], idx_v)

        def issue_gather(j, b):
            pltpu.async_copy(table_hbm.at[idx_v.at[j]], bufs[b], gs[b])

        def wait_gather(b):
            pltpu.make_async_copy(out_hbm.at[pl.ds(0, _CHUNK)], bufs[b], gs[b]).wait()

        def issue_write(j, b):
            pltpu.async_copy(bufs[b], out_hbm.at[pl.ds(base + j * _CHUNK, _CHUNK)], ws[b])

        def wait_write(b):
            pltpu.make_async_copy(bufs[b], out_hbm.at[pl.ds(0, _CHUNK)], ws[b]).wait()

        for b in range(_NBUF):
            issue_gather(b, b)

        @pl.loop(0, nchunk_w // _NBUF - 1)
        def _(kk):
            j0 = kk * _NBUF
            for b in range(_NBUF):
                wait_gather(b)
                issue_write(j0 + b, b)
                wait_write(b)
                issue_gather(j0 + _NBUF + b, b)

        j0 = nchunk_w - _NBUF
        for b in range(_NBUF):
            wait_gather(b)
            issue_write(j0 + b, b)
        for b in range(_NBUF):
            wait_write(b)

    return k(idx, table)


def kernel(input, table):
    seq, batch = input.shape
    n = seq * batch
    idx = input.reshape(n // _CHUNK, _CHUNK).astype(jnp.int32)
    table_p = jnp.pad(table.astype(jnp.float32), ((0, 0), (0, _EMBP - _EMB)))
    out = _sc_gather(idx, table_p, n)
    return out[:, :_EMB].reshape(seq, batch, _EMB)


# speedup vs baseline: 5.3689x; 1.2505x over previous
import functools

import jax
import jax.numpy as jnp
from jax import lax
from jax.experimental import pallas as pl
from jax.experimental.pallas import tpu as pltpu
from jax.experimental.pallas import tpu_sc as plsc

_EMB = 64
_EMBP = 128
_NW = 32
_CHUNK = 128
_NBUF = 5


@functools.partial(jax.jit, static_argnames=("n",))
def _sc_gather(idx, table2, n):
    nchunk_w = n // (_NW * _CHUNK)
    assert nchunk_w % _NBUF == 0 and nchunk_w >= 2 * _NBUF
    mesh = plsc.VectorSubcoreMesh(core_axis_name="c", subcore_axis_name="s")

    @functools.partial(
        pl.kernel,
        out_type=jax.ShapeDtypeStruct((n, _EMBP), jnp.float32),
        mesh=mesh,
        scratch_types=[
            pltpu.VMEM((nchunk_w, _CHUNK), jnp.int32),
            [pltpu.VMEM((_CHUNK, _EMB), jnp.float32)] * _NBUF,
            [pltpu.SemaphoreType.DMA] * _NBUF,
            [pltpu.SemaphoreType.DMA] * _NBUF,
        ],
        compiler_params=pltpu.CompilerParams(use_tc_tiling_on_sc=False),
    )
    def k(idx_hbm, table_hbm, out_hbm, idx_v, bufs, gs, ws):
        wid = lax.axis_index("s") * 2 + lax.axis_index("c")
        base = wid * (nchunk_w * _CHUNK)
        pltpu.sync_copy(idx_hbm.at[pl.ds(wid * nchunk_w, nchunk_w)], idx_v)

        def out_slice(j):
            return out_hbm.at[pl.ds(base + j * _CHUNK, _CHUNK), pl.ds(0, _EMB)]

        def issue_gather(j, b):
            pltpu.async_copy(table_hbm.at[idx_v.at[j]], bufs[b], gs[b])

        def wait_gather(b):
            pltpu.make_async_copy(table_hbm.at[pl.ds(0, _CHUNK)], bufs[b], gs[b]).wait()

        def issue_write(j, b):
            pltpu.async_copy(bufs[b], out_slice(j), ws[b])

        def wait_write(b):
            pltpu.make_async_copy(bufs[b], out_slice(0), ws[b]).wait()

        for b in range(_NBUF):
            issue_gather(b, b)

        @pl.loop(0, nchunk_w // _NBUF - 1)
        def _(kk):
            j0 = kk * _NBUF
            for b in range(_NBUF):
                wait_gather(b)
                issue_write(j0 + b, b)
                wait_write(b)
                issue_gather(j0 + _NBUF + b, b)

        j0 = nchunk_w - _NBUF
        for b in range(_NBUF):
            wait_gather(b)
            issue_write(j0 + b, b)
        for b in range(_NBUF):
            wait_write(b)

    return k(idx, table2)


def kernel(input, table):
    seq, batch = input.shape
    n = seq * batch
    idx = (input.astype(jnp.int32) * 2).reshape(n // _CHUNK, _CHUNK)
    table_p = jnp.pad(table.astype(jnp.float32), ((0, 0), (0, _EMBP - _EMB)))
    table2 = table_p.reshape(2 * table.shape[0], _EMB)
    out = _sc_gather(idx, table2, n)
    return out[:, :_EMB].reshape(seq, batch, _EMB)
